# Initial kernel scaffold; baseline (speedup 1.0000x reference)
import jax
import jax.numpy as jnp
from jax.experimental import pallas as pl


def kernel(contents, W_u, b_u, W_h, b_h):
    raise NotImplementedError("write your pallas kernel here")


# speedup vs baseline: 27.8547x; 27.8547x over previous
import numpy as np
import jax
import jax.numpy as jnp
from jax.experimental import pallas as pl
from jax.experimental.pallas import tpu as pltpu

_B = 128
_DEPTH = 11
_NF = 128
_NH = 64
_LEVEL_SIZES = [_B * (2 ** j) for j in range(_DEPTH + 1)]
_OFFSETS = [int(x) for x in np.concatenate([[0], np.cumsum(_LEVEL_SIZES)])]
_G = 8
_NLEV = _DEPTH + 1


def _grnn_kernel(c_hbm, wu_ref, bu_ref, wh_ref, bh_ref, out_ref, *rest):
    bufs = rest[:_NLEV]
    sems = rest[_NLEV:2 * _NLEV]
    emb_buf = rest[2 * _NLEV]
    g = pl.program_id(0)
    slot = jax.lax.rem(g, 2)
    nslot = jax.lax.rem(g + 1, 2)

    def _copy(step, sl, idx):
        j = _DEPTH - idx
        rows = _G << j
        start = _OFFSETS[j] + step * rows
        return pltpu.make_async_copy(
            c_hbm.at[pl.ds(start, rows)], bufs[idx].at[sl], sems[idx].at[sl]
        )

    @pl.when(g == 0)
    def _():
        for idx in range(_NLEV):
            _copy(g, slot, idx).start()

    @pl.when(g + 1 < pl.num_programs(0))
    def _():
        for idx in range(_NLEV):
            _copy(g + 1, nslot, idx).start()

    wu = wu_ref[:]
    bu = bu_ref[:]
    wh_l = wh_ref[:_NH, :]
    wh_r = wh_ref[_NH : 2 * _NH, :]
    wh_u = wh_ref[2 * _NH :, :]
    bh = bh_ref[:]

    _copy(g, slot, 0).wait()
    emb = jnp.tanh(
        jnp.dot(bufs[0][slot], wu, preferred_element_type=jnp.float32) + bu
    )
    pairs_view = emb_buf.reshape(_G << (_DEPTH - 1), 2, _NH)
    for idx in range(1, _NLEV):
        j = _DEPTH - idx
        n = _G << j
        emb_buf[pl.ds(0, 2 * n), :] = emb
        _copy(g, slot, idx).wait()
        c = bufs[idx][slot]
        u = jnp.tanh(jnp.dot(c, wu, preferred_element_type=jnp.float32) + bu)
        h_l = pairs_view[pl.ds(0, n), 0, :]
        h_r = pairs_view[pl.ds(0, n), 1, :]
        emb = jnp.tanh(
            jnp.dot(h_l, wh_l, preferred_element_type=jnp.float32)
            + jnp.dot(h_r, wh_r, preferred_element_type=jnp.float32)
            + jnp.dot(u, wh_u, preferred_element_type=jnp.float32)
            + bh
        )
    out_ref[:] = emb


@jax.jit
def kernel(contents, W_u, b_u, W_h, b_h):
    grid = (_B // _G,)
    scratch = [
        pltpu.VMEM((2, _G << (_DEPTH - idx), _NF), jnp.float32)
        for idx in range(_NLEV)
    ] + [pltpu.SemaphoreType.DMA((2,))] * _NLEV + [
        pltpu.VMEM((_G << _DEPTH, _NH), jnp.float32)
    ]
    out = pl.pallas_call(
        _grnn_kernel,
        grid=grid,
        in_specs=[
            pl.BlockSpec(memory_space=pltpu.MemorySpace.HBM),
            pl.BlockSpec((_NF, _NH), lambda g: (0, 0)),
            pl.BlockSpec((1, _NH), lambda g: (0, 0)),
            pl.BlockSpec((3 * _NH, _NH), lambda g: (0, 0)),
            pl.BlockSpec((1, _NH), lambda g: (0, 0)),
        ],
        out_specs=pl.BlockSpec((_G, _NH), lambda g: (g, 0)),
        out_shape=jax.ShapeDtypeStruct((_B, _NH), jnp.float32),
        scratch_shapes=scratch,
        compiler_params=pltpu.CompilerParams(
            dimension_semantics=("arbitrary",),
        ),
    )(contents, W_u, b_u.reshape(1, _NH), W_h, b_h.reshape(1, _NH))
    return out
